# Initial kernel scaffold; baseline (speedup 1.0000x reference)
import jax
import jax.numpy as jnp
from jax.experimental import pallas as pl


def kernel(sparse_features, dense_features, wide_sparse_table, wide_dense_W, wide_dense_b, deep_table, W1, b1, W2, b2, W3, b3, Wc, bc):
    raise NotImplementedError("write your pallas kernel here")


# speedup vs baseline: 14.0662x; 14.0662x over previous
import functools

import jax
import jax.numpy as jnp
from jax import lax
from jax.experimental import pallas as pl
from jax.experimental.pallas import tpu as pltpu
from jax.experimental.pallas import tpu_sc as plsc

_B, _F, _V, _D, _ND = 16384, 26, 1000000, 16, 13
_NW = 32
_C = 128
_N = _B * _F
_CPW = _N // (_NW * _C)


def _sc_gather(idx2d, deep_table, wide_tab):
    mesh = plsc.VectorSubcoreMesh(core_axis_name="c", subcore_axis_name="s")
    nc = 2

    @functools.partial(
        pl.kernel,
        out_type=(
            jax.ShapeDtypeStruct((_N, _D), jnp.float32),
            jax.ShapeDtypeStruct((_N,), jnp.float32),
        ),
        mesh=mesh,
        compiler_params=pltpu.CompilerParams(use_tc_tiling_on_sc=False),
        scratch_types=[
            pltpu.VMEM((_CPW, _C), jnp.int32),
            pltpu.VMEM((_C, _D), jnp.float32),
            pltpu.VMEM((_C,), jnp.float32),
            pltpu.SemaphoreType.DMA,
            pltpu.SemaphoreType.DMA,
        ],
    )
    def k(idx_hbm, deep_hbm, wide_hbm, emb_out, wide_out,
          idx_v, rows_v, wvals_v, sem_d, sem_w):
        wid = lax.axis_index("s") * nc + lax.axis_index("c")
        row0 = wid * _CPW
        pltpu.sync_copy(idx_hbm.at[pl.ds(row0, _CPW)], idx_v)

        def chunk(j, _):
            cd = pltpu.async_copy(deep_hbm.at[idx_v.at[j]], rows_v, sem_d)
            cw = pltpu.async_copy(wide_hbm.at[idx_v.at[j]], wvals_v, sem_w)
            cd.wait()
            cw.wait()
            base = (row0 + j) * _C
            pltpu.sync_copy(rows_v, emb_out.at[pl.ds(base, _C)])
            pltpu.sync_copy(wvals_v, wide_out.at[pl.ds(base, _C)])
            return 0

        lax.fori_loop(0, _CPW, chunk, 0)

    return k(idx2d, deep_table, wide_tab)


def _mlp_body(emb_ref, dense_ref, wvals_ref, w1e_ref, w1d_ref, b1_ref,
              w2_ref, b2_ref, w3_ref, b3_ref, wcd_ref, wdw_ref, scal_ref,
              out_ref):
    f32 = jnp.float32
    h1 = jnp.dot(emb_ref[...], w1e_ref[...], preferred_element_type=f32)
    h1 = h1 + jnp.dot(dense_ref[...], w1d_ref[...], preferred_element_type=f32)
    h1 = jnp.maximum(h1 + b1_ref[...], 0.0)
    h2 = jnp.maximum(jnp.dot(h1, w2_ref[...], preferred_element_type=f32) + b2_ref[...], 0.0)
    h3 = jnp.maximum(jnp.dot(h2, w3_ref[...], preferred_element_type=f32) + b3_ref[...], 0.0)
    wc0 = scal_ref[0, 0]
    bc = scal_ref[0, 1]
    wdb = scal_ref[0, 2]
    wide = (jnp.sum(wvals_ref[...], axis=1, keepdims=True)
            + jnp.sum(dense_ref[...] * wdw_ref[...], axis=1, keepdims=True) + wdb)
    deep_out = jnp.sum(h3 * wcd_ref[...], axis=1, keepdims=True)
    out_ref[...] = wc0 * wide + deep_out + bc


def _mlp(emb, dense, wvals, w1e, w1d, b1, w2, b2, w3, b3, wcd, wdw, scal):
    blk = 1024
    grid = (_B // blk,)
    full = lambda a: pl.BlockSpec(a.shape, lambda i: (0, 0))
    return pl.pallas_call(
        _mlp_body,
        grid=grid,
        in_specs=[
            pl.BlockSpec((blk, _F * _D), lambda i: (i, 0)),
            pl.BlockSpec((blk, _ND), lambda i: (i, 0)),
            pl.BlockSpec((blk, _F), lambda i: (i, 0)),
            full(w1e), full(w1d), full(b1), full(w2), full(b2),
            full(w3), full(b3), full(wcd), full(wdw), full(scal),
        ],
        out_specs=pl.BlockSpec((blk, 1), lambda i: (i, 0)),
        out_shape=jax.ShapeDtypeStruct((_B, 1), jnp.float32),
    )(emb, dense, wvals, w1e, w1d, b1, w2, b2, w3, b3, wcd, wdw, scal)


def kernel(sparse_features, dense_features, wide_sparse_table, wide_dense_W,
           wide_dense_b, deep_table, W1, b1, W2, b2, W3, b3, Wc, bc):
    idx2d = sparse_features.astype(jnp.int32).reshape(_N // _C, _C)
    wide_tab = wide_sparse_table.reshape(-1)
    emb_flat, wide_flat = _sc_gather(idx2d, deep_table, wide_tab)
    emb = emb_flat.reshape(_B, _F * _D)
    wvals = wide_flat.reshape(_B, _F)

    ed = _F * _D
    w1e = W1[:, :ed].T
    w1d = W1[:, ed:].T
    scal = jnp.stack([Wc[0, 0], bc[0], wide_dense_b[0]]).reshape(1, 3)
    return _mlp(emb, dense_features, wvals,
                w1e, w1d, b1.reshape(1, -1),
                W2.T, b2.reshape(1, -1), W3.T, b3.reshape(1, -1),
                Wc[:, 1:], wide_dense_W, scal)
